# Initial kernel scaffold; baseline (speedup 1.0000x reference)
import jax
import jax.numpy as jnp
from jax.experimental import pallas as pl


def kernel(x, table):
    raise NotImplementedError("write your pallas kernel here")


# speedup vs baseline: 1.7720x; 1.7720x over previous
import functools
import math

import jax
import jax.numpy as jnp
import numpy as np
from jax import lax
from jax.experimental import pallas as pl
from jax.experimental.pallas import tpu as pltpu
from jax.experimental.pallas import tpu_sc as plsc

_VOCAB = 1000000
_D = 64
_BATCH = 16384
_SEQ = 50

_NW = 32
_ROWS = _BATCH * _SEQ
_PER_W = _ROWS // _NW
_C = 400
_NCHUNK = _PER_W // _C
_K = 5
_SUB = _C // _K
_SCALE = 8.0


def _pos_encoding_block():
    positions = np.arange(_SEQ)[:, np.newaxis]
    div_term = np.exp(np.arange(0, _D, 2) * -(np.log(10000.0) / _D))
    angle_rads = positions * div_term
    pe = np.zeros((_SEQ, _D), dtype=np.float32)
    pe[:, 0::2] = np.sin(angle_rads)
    pe[:, 1::2] = np.cos(angle_rads)
    return np.tile(pe, (_C // _SEQ, 1))


_PE_BLOCK = _pos_encoding_block()


def _make_sc_kernel():
    mesh = plsc.VectorSubcoreMesh(core_axis_name="c", subcore_axis_name="s")

    @functools.partial(
        pl.kernel,
        out_type=jax.ShapeDtypeStruct((_ROWS, _D), jnp.float32),
        mesh=mesh,
        compiler_params=pltpu.CompilerParams(use_tc_tiling_on_sc=False),
        scratch_types=[
            pltpu.VMEM((_NCHUNK, _K, _SUB), jnp.int32),
            pltpu.VMEM((2, _C, _D), jnp.float32),
            pltpu.VMEM((_C, _D), jnp.float32),
            pltpu.SemaphoreType.DMA,
            pltpu.SemaphoreType.DMA,
        ],
    )
    def sc_kernel(x_hbm, pe_hbm, table_hbm, out_hbm, idx_v, rows_v, pe_v, g0, g1):
        wid = lax.axis_index("s") * 2 + lax.axis_index("c")

        pltpu.sync_copy(x_hbm.at[wid], idx_v)
        pltpu.sync_copy(pe_hbm, pe_v)

        sems = (g0, g1)

        def fire(c, b):
            for k in range(_K):
                pltpu.async_copy(
                    table_hbm.at[idx_v.at[c, k]],
                    rows_v.at[b, pl.ds(_SUB * k, _SUB)],
                    sems[b],
                )

        def finish(c, b):
            for k in range(_K):
                pltpu.make_async_copy(
                    table_hbm.at[idx_v.at[c, k]],
                    rows_v.at[b, pl.ds(_SUB * k, _SUB)],
                    sems[b],
                ).wait()

            def body(i, carry):
                for j in range(_D // 16):
                    sl = pl.ds(16 * j, 16)
                    rows_v[b, i, sl] = rows_v[b, i, sl] * _SCALE + pe_v[i, sl]
                return carry

            lax.fori_loop(0, _C, body, 0)
            row0 = (wid * _NCHUNK + c) * _C
            pltpu.sync_copy(rows_v.at[b], out_hbm.at[pl.ds(row0, _C)])

        fire(0, 0)

        def loop_body(c2, carry):
            c0 = 2 * c2

            fire(c0 + 1, 1)
            finish(c0, 0)

            @pl.when(c0 + 2 < _NCHUNK)
            def _():
                fire(c0 + 2, 0)

            finish(c0 + 1, 1)
            return carry

        lax.fori_loop(0, _NCHUNK // 2, loop_body, 0)

    return sc_kernel


_sc_kernel = _make_sc_kernel()


@jax.jit
def kernel(x, table):
    x_r = x.reshape(_NW, _NCHUNK, _K, _SUB)
    pe = jnp.asarray(_PE_BLOCK)
    out = _sc_kernel(x_r, pe, table)
    return out.reshape(_BATCH, _SEQ, _D)
